# baseline (speedup 1.0000x reference)
import functools

import jax
import jax.numpy as jnp
from jax import lax
from jax.experimental import pallas as pl
from jax.experimental.pallas import tpu as pltpu
from jax.experimental.pallas import tpu_sc as plsc

_NC = 2
_NS = 16
_L = 16


def _sc_hist_gather(edge_index, node_type, out_idx):
    n = node_type.shape[0]
    e = edge_index.shape[1]
    k = out_idx.shape[0]
    nw = _NC * _NS
    assert e % 128 == 0
    rows = e // 128
    base_rows = rows // nw
    extra = rows - base_rows * nw
    main_e = base_rows * 128
    chunks = base_rows + (1 if extra else 0)
    epad = chunks * 128
    assert k % _NS == 0
    kp = k // _NS
    assert kp % _L == 0
    dump = 3 * k
    tp = 3 * k + 16 * _NS
    zslice = tp // _NS
    assert zslice % _L == 0
    sslice = -(-n // (_NS * _L)) * _L
    smp = sslice * _NS
    qcap = epad + 128

    mesh = plsc.VectorSubcoreMesh(core_axis_name="c", subcore_axis_name="s")

    @functools.partial(
        pl.kernel,
        out_type=[
            jax.ShapeDtypeStruct((_NC, 8, k), jnp.float32),
            jax.ShapeDtypeStruct((k,), jnp.int32),
        ],
        mesh=mesh,
        compiler_params=pltpu.CompilerParams(needs_layout_passes=False),
        scratch_types=[
            pltpu.VMEM((2, epad), jnp.int32),
            pltpu.VMEM((n,), jnp.int32),
            pltpu.VMEM((smp,), jnp.int32),
            pltpu.VMEM((k,), jnp.int32),
            pltpu.VMEM((qcap,), jnp.int32),
            pltpu.VMEM((qcap // 128, 128), jnp.int32),
            pltpu.VMEM((128,), jnp.float32),
            pltpu.VMEM((8, 128), jnp.int32),
            pltpu.VMEM((8, 128), jnp.float32),
            pltpu.VMEM((kp,), jnp.int32),
            pltpu.VMEM((zslice,), jnp.float32),
            pltpu.VMEM_SHARED((tp,), jnp.float32),
            pltpu.SemaphoreType.DMA,
            pltpu.SemaphoreType.DMA,
            pltpu.SemaphoreType.DMA,
        ],
    )
    def hist(edge_hbm, nt_hbm, oi_hbm, slmi_hbm, cnt_out, ty_out,
             e2_v, nt_v, slm_v, oi_v, idx_q, stage2d, ones_v,
             idxg, stg, stgt, zb, counts_sh, sem_in, sem_sc, sem_g):
        cid = lax.axis_index("c")
        sid = lax.axis_index("s")
        wid = cid * _NS + sid

        zeros16f = jnp.zeros((_L,), jnp.float32)
        ones16f = jnp.ones((_L,), jnp.float32)
        zeros16i = jnp.zeros((_L,), jnp.int32)
        dump16 = jnp.full((_L,), dump, jnp.int32)
        iota = lax.iota(jnp.int32, _L)

        col0 = pl.multiple_of(wid * main_e, 128)
        cp_edge = pltpu.async_copy(edge_hbm.at[:, pl.ds(col0, main_e)],
                                   e2_v.at[:, pl.ds(0, main_e)], sem_in)
        cp_nt = pltpu.async_copy(nt_hbm, nt_v, sem_in)
        cp_oi = pltpu.async_copy(oi_hbm, oi_v, sem_g)
        cp_slm = pltpu.async_copy(slmi_hbm, slm_v, sem_g)

        def zb_body(i, _):
            zb[pl.ds(i * _L, _L)] = zeros16f
            return 0
        lax.fori_loop(0, zslice // _L, zb_body, 0)
        pltpu.sync_copy(zb, counts_sh.at[pl.ds(sid * zslice, zslice)])

        for u in range(128 // _L):
            ones_v[pl.ds(u * _L, _L)] = ones16f

        if extra:
            for r in range(2):
                for u in range(128 // _L):
                    e2_v[r, pl.ds(main_e + u * _L, _L)] = zeros16i

            @pl.when(wid < extra)
            def _():
                tcol = pl.multiple_of((nw * base_rows + wid) * 128, 128)
                pltpu.sync_copy(edge_hbm.at[:, pl.ds(tcol, 128)],
                                e2_v.at[:, pl.ds(main_e, 128)])

        cp_oi.wait()
        cp_slm.wait()

        def slot_body(v, _):
            ov = oi_v[pl.ds(v * _L, _L)]
            plsc.store_scatter(slm_v, [ov], (v * _L + iota) * 3)
            return 0
        lax.fori_loop(0, k // _L, slot_body, 0)

        cp_edge.wait()
        cp_nt.wait()
        plsc.subcore_barrier()

        def compact(c, keep_scale):
            for u in range(128 // _L):
                base = c * 128 + u * _L
                s = e2_v[0, pl.ds(base, _L)]
                d = e2_v[1, pl.ds(base, _L)]
                t = plsc.load_gather(nt_v, [s])
                sl3 = plsc.load_gather(slm_v, [d])
                keep = (sl3 < dump).astype(jnp.int32) * keep_scale
                idxv = jnp.where(keep > 0, sl3 + t, dump16)
                idx_q[pl.ds(base, _L)] = idxv
            return 0

        lax.fori_loop(0, base_rows, lambda c, _: compact(c, 1), 0)
        if extra:
            compact(base_rows, (wid < extra).astype(jnp.int32))
        nchunks = chunks

        DEPTH = 8
        maxc = qcap // 128

        def fire(w):
            pltpu.async_copy(ones_v, counts_sh.at[stage2d.at[w]], sem_sc,
                             add=True)

        def drain(w):
            pltpu.make_async_copy(ones_v, counts_sh.at[stage2d.at[w]],
                                  sem_sc).wait()

        def s_body(c, _):
            @pl.when(c < nchunks)
            def _():
                for u in range(128 // _L):
                    stage2d[c, pl.ds(u * _L, _L)] = idx_q[
                        pl.ds(c * 128 + u * _L, _L)]
                fire(c)

            @pl.when((c >= DEPTH) & (c < nchunks))
            def _():
                drain(c - DEPTH)
            return 0
        lax.fori_loop(0, maxc, s_body, 0)
        for i in range(DEPTH):
            w = nchunks - DEPTH + i
            if isinstance(nchunks, int):
                if w >= 0:
                    drain(w)
            else:
                @pl.when(w >= 0)
                def _():
                    drain(w)

        plsc.subcore_barrier()

        for v in range(kp // _L):
            o = plsc.load_gather(oi_v, [sid * kp + v * _L + iota])
            t = plsc.load_gather(nt_v, [o])
            stgt[pl.ds(v * _L, _L)] = t
            sl3o = plsc.load_gather(slm_v, [o])
            for j in range(8):
                if j < 3:
                    idxg[j, pl.ds(v * _L, _L)] = sl3o + j
                else:
                    idxg[j, pl.ds(v * _L, _L)] = dump16
        for j in range(8):
            pltpu.async_copy(counts_sh.at[idxg.at[j]], stg.at[j], sem_g)
        for j in range(8):
            pltpu.make_async_copy(counts_sh.at[idxg.at[j]], stg.at[j],
                                  sem_g).wait()
        pltpu.sync_copy(stg, cnt_out.at[cid, :, pl.ds(sid * kp, kp)])

        @pl.when(cid == 0)
        def _():
            pltpu.sync_copy(stgt, ty_out.at[pl.ds(sid * kp, kp)])

    slm_init = jnp.full((smp,), dump, jnp.int32)
    return hist(edge_index, node_type, out_idx, slm_init)


def _tc_readout(init_features, W_init, b_init, W_agg, W_self, b_gnn,
                W1, b1, W2, b2, W3, b3, cnt8, ty):
    k = ty.shape[0]
    h_dim = W_agg.shape[0]

    def body(if_ref, wi_ref, bi_ref, wa_ref, ws_ref, bg_ref,
             w1_ref, b1_ref, w2_ref, b2_ref, w3_ref, b3_ref,
             cnt_ref, ty_ref, out_ref):
        ce_rows = [if_ref[t:t + 1, :] @ wi_ref[t] + bi_ref[t:t + 1, :]
                   for t in range(3)]
        ce8 = jnp.concatenate(ce_rows + [jnp.zeros((5, h_dim), jnp.float32)],
                              axis=0)
        m8 = ce8 @ wa_ref[...]
        s8 = ce8 @ ws_ref[...] + bg_ref[...]
        cnt = cnt_ref[0] + cnt_ref[1]
        oh = (lax.broadcasted_iota(jnp.int32, (8, k), 0)
              == ty_ref[...][None, :]).astype(jnp.float32)
        x = jnp.concatenate([cnt, oh], axis=0)
        w0 = jnp.concatenate([m8, s8], axis=0)
        h = lax.dot_general(x, w0, (((0,), (0,)), ((), ())),
                            precision=lax.Precision.HIGHEST,
                            preferred_element_type=jnp.float32)
        h = jnp.maximum(h, 0.0)
        h = jnp.maximum(h @ w1_ref[...] + b1_ref[...], 0.0)
        h = jnp.maximum(h @ w2_ref[...] + b2_ref[...], 0.0)
        z = h @ w3_ref[...] + b3_ref[...]
        out_ref[...] = jax.nn.sigmoid(z)

    return pl.pallas_call(
        body,
        out_shape=jax.ShapeDtypeStruct((k, 1), jnp.float32),
    )(init_features, W_init, b_init, W_agg, W_self,
      b_gnn.reshape(1, h_dim), W1, b1.reshape(1, h_dim), W2,
      b2.reshape(1, h_dim), W3, b3.reshape(1, 1), cnt8, ty)


def kernel(init_features, W_init, b_init, W_agg, W_self, b_gnn,
           W1, b1, W2, b2, W3, b3, node_type, edge_index, out_idx):
    k = out_idx.shape[0]
    nt = node_type.astype(jnp.int32)
    ei = edge_index.astype(jnp.int32)
    oi = out_idx.astype(jnp.int32)
    cnt, ty = _sc_hist_gather(ei, nt, oi)
    out2d = _tc_readout(init_features.astype(jnp.float32),
                        W_init.astype(jnp.float32),
                        b_init.astype(jnp.float32),
                        W_agg.astype(jnp.float32),
                        W_self.astype(jnp.float32),
                        b_gnn.astype(jnp.float32),
                        W1.astype(jnp.float32), b1.astype(jnp.float32),
                        W2.astype(jnp.float32), b2.astype(jnp.float32),
                        W3.astype(jnp.float32), b3.astype(jnp.float32),
                        cnt, ty)
    return out2d.reshape(k)


# speedup vs baseline: 3.9066x; 3.9066x over previous
import functools

import jax
import jax.numpy as jnp
from jax import lax
from jax.experimental import pallas as pl
from jax.experimental.pallas import tpu as pltpu
from jax.experimental.pallas import tpu_sc as plsc

_NC = 2
_NS = 16
_L = 16


def _sc_hist_gather(edge_index, node_type, out_idx):
    n = node_type.shape[0]
    e = edge_index.shape[1]
    k = out_idx.shape[0]
    nw = _NC * _NS
    assert e % 128 == 0
    rows = e // 128
    base_rows = rows // nw
    extra = rows - base_rows * nw
    main_e = base_rows * 128
    chunks = base_rows + (1 if extra else 0)
    epad = chunks * 128
    assert k % _NS == 0
    kp = k // _NS
    assert kp % _L == 0
    dump = 3 * k
    tp = 3 * k + 16 * _NS
    zslice = tp // _NS
    assert zslice % _L == 0
    sslice = -(-n // (_NS * _L)) * _L
    smp = sslice * _NS
    qcap = epad + 128

    mesh = plsc.VectorSubcoreMesh(core_axis_name="c", subcore_axis_name="s")

    @functools.partial(
        pl.kernel,
        out_type=[
            jax.ShapeDtypeStruct((_NC, 8, k), jnp.float32),
            jax.ShapeDtypeStruct((k,), jnp.int32),
        ],
        mesh=mesh,
        compiler_params=pltpu.CompilerParams(needs_layout_passes=False),
        scratch_types=[
            pltpu.VMEM((2, epad), jnp.int32),
            pltpu.VMEM((n,), jnp.int32),
            pltpu.VMEM((smp,), jnp.int32),
            pltpu.VMEM((k,), jnp.int32),
            pltpu.VMEM((qcap,), jnp.int32),
            pltpu.VMEM((qcap // 128, 128), jnp.int32),
            pltpu.VMEM((128,), jnp.float32),
            pltpu.VMEM((8, 128), jnp.int32),
            pltpu.VMEM((8, 128), jnp.float32),
            pltpu.VMEM((kp,), jnp.int32),
            pltpu.VMEM((zslice,), jnp.float32),
            pltpu.VMEM_SHARED((tp,), jnp.float32),
            pltpu.SemaphoreType.DMA,
            pltpu.SemaphoreType.DMA,
            pltpu.SemaphoreType.DMA,
        ],
    )
    def hist(edge_hbm, nt_hbm, oi_hbm, slmi_hbm, cnt_out, ty_out,
             e2_v, nt_v, slm_v, oi_v, idx_q, stage2d, ones_v,
             idxg, stg, stgt, zb, counts_sh, sem_in, sem_sc, sem_g):
        cid = lax.axis_index("c")
        sid = lax.axis_index("s")
        wid = cid * _NS + sid

        zeros16f = jnp.zeros((_L,), jnp.float32)
        ones16f = jnp.ones((_L,), jnp.float32)
        zeros16i = jnp.zeros((_L,), jnp.int32)
        dump16 = jnp.full((_L,), dump, jnp.int32)
        iota = lax.iota(jnp.int32, _L)

        col0 = pl.multiple_of(wid * main_e, 128)
        cp_edge = pltpu.async_copy(edge_hbm.at[:, pl.ds(col0, main_e)],
                                   e2_v.at[:, pl.ds(0, main_e)], sem_in)
        cp_nt = pltpu.async_copy(nt_hbm, nt_v, sem_in)
        cp_oi = pltpu.async_copy(oi_hbm, oi_v, sem_g)
        cp_slm = pltpu.async_copy(slmi_hbm, slm_v, sem_g)

        def zb_body(i, _):
            zb[pl.ds(i * _L, _L)] = zeros16f
            return 0
        lax.fori_loop(0, zslice // _L, zb_body, 0)
        pltpu.sync_copy(zb, counts_sh.at[pl.ds(sid * zslice, zslice)])

        for u in range(128 // _L):
            ones_v[pl.ds(u * _L, _L)] = ones16f

        if extra:
            for r in range(2):
                for u in range(128 // _L):
                    e2_v[r, pl.ds(main_e + u * _L, _L)] = zeros16i

            @pl.when(wid < extra)
            def _():
                tcol = pl.multiple_of((nw * base_rows + wid) * 128, 128)
                pltpu.sync_copy(edge_hbm.at[:, pl.ds(tcol, 128)],
                                e2_v.at[:, pl.ds(main_e, 128)])

        cp_oi.wait()
        cp_slm.wait()

        def slot_body(v, _):
            ov = oi_v[pl.ds(v * _L, _L)]
            plsc.store_scatter(slm_v, [ov], (v * _L + iota) * 3)
            return 0
        lax.fori_loop(0, k // _L, slot_body, 0)

        cp_edge.wait()
        cp_nt.wait()
        plsc.subcore_barrier()

        def compact(c, off, keep_scale):
            for u in range(128 // _L):
                base = c * 128 + u * _L
                s = e2_v[0, pl.ds(base, _L)]
                d = e2_v[1, pl.ds(base, _L)]
                t = plsc.load_gather(nt_v, [s])
                sl3 = plsc.load_gather(slm_v, [d])
                keep = (sl3 < dump).astype(jnp.int32) * keep_scale
                csum = jnp.sum(keep)
                plsc.store_compressed(idx_q.at[pl.ds(off, _L)], sl3 + t,
                                      mask=keep > 0)
                off = off + csum
            return off

        off = lax.fori_loop(0, base_rows, lambda c, o: compact(c, o, 1), 0)
        if extra:
            off = compact(base_rows, off, (wid < extra).astype(jnp.int32))
        for u in range(128 // _L):
            idx_q[pl.ds(off + u * _L, _L)] = dump16
        nchunks = lax.shift_right_logical(off, 7) + 1

        DEPTH = 8
        maxc = qcap // 128

        def fire(w):
            pltpu.async_copy(ones_v, counts_sh.at[stage2d.at[w]], sem_sc,
                             add=True)

        def drain(w):
            pltpu.make_async_copy(ones_v, counts_sh.at[stage2d.at[w]],
                                  sem_sc).wait()

        def s_body(c, _):
            @pl.when(c < nchunks)
            def _():
                for u in range(128 // _L):
                    stage2d[c, pl.ds(u * _L, _L)] = idx_q[
                        pl.ds(c * 128 + u * _L, _L)]
                fire(c)

            @pl.when((c >= DEPTH) & (c < nchunks))
            def _():
                drain(c - DEPTH)
            return 0
        lax.fori_loop(0, maxc, s_body, 0)
        for i in range(DEPTH):
            w = nchunks - DEPTH + i
            if isinstance(nchunks, int):
                if w >= 0:
                    drain(w)
            else:
                @pl.when(w >= 0)
                def _():
                    drain(w)

        plsc.subcore_barrier()

        for v in range(kp // _L):
            o = plsc.load_gather(oi_v, [sid * kp + v * _L + iota])
            t = plsc.load_gather(nt_v, [o])
            stgt[pl.ds(v * _L, _L)] = t
            sl3o = plsc.load_gather(slm_v, [o])
            for j in range(8):
                if j < 3:
                    idxg[j, pl.ds(v * _L, _L)] = sl3o + j
                else:
                    idxg[j, pl.ds(v * _L, _L)] = dump16
        for j in range(8):
            pltpu.async_copy(counts_sh.at[idxg.at[j]], stg.at[j], sem_g)
        for j in range(8):
            pltpu.make_async_copy(counts_sh.at[idxg.at[j]], stg.at[j],
                                  sem_g).wait()
        pltpu.sync_copy(stg, cnt_out.at[cid, :, pl.ds(sid * kp, kp)])

        @pl.when(cid == 0)
        def _():
            pltpu.sync_copy(stgt, ty_out.at[pl.ds(sid * kp, kp)])

    slm_init = jnp.full((smp,), dump, jnp.int32)
    return hist(edge_index, node_type, out_idx, slm_init)


def _tc_readout(init_features, W_init, b_init, W_agg, W_self, b_gnn,
                W1, b1, W2, b2, W3, b3, cnt8, ty):
    k = ty.shape[0]
    h_dim = W_agg.shape[0]

    def body(if_ref, wi_ref, bi_ref, wa_ref, ws_ref, bg_ref,
             w1_ref, b1_ref, w2_ref, b2_ref, w3_ref, b3_ref,
             cnt_ref, ty_ref, out_ref):
        ce_rows = [if_ref[t:t + 1, :] @ wi_ref[t] + bi_ref[t:t + 1, :]
                   for t in range(3)]
        ce8 = jnp.concatenate(ce_rows + [jnp.zeros((5, h_dim), jnp.float32)],
                              axis=0)
        m8 = ce8 @ wa_ref[...]
        s8 = ce8 @ ws_ref[...] + bg_ref[...]
        cnt = cnt_ref[0] + cnt_ref[1]
        oh = (lax.broadcasted_iota(jnp.int32, (8, k), 0)
              == ty_ref[...][None, :]).astype(jnp.float32)
        x = jnp.concatenate([cnt, oh], axis=0)
        w0 = jnp.concatenate([m8, s8], axis=0)
        h = lax.dot_general(x, w0, (((0,), (0,)), ((), ())),
                            precision=lax.Precision.HIGHEST,
                            preferred_element_type=jnp.float32)
        h = jnp.maximum(h, 0.0)
        h = jnp.maximum(h @ w1_ref[...] + b1_ref[...], 0.0)
        h = jnp.maximum(h @ w2_ref[...] + b2_ref[...], 0.0)
        z = h @ w3_ref[...] + b3_ref[...]
        out_ref[...] = jax.nn.sigmoid(z)

    return pl.pallas_call(
        body,
        out_shape=jax.ShapeDtypeStruct((k, 1), jnp.float32),
    )(init_features, W_init, b_init, W_agg, W_self,
      b_gnn.reshape(1, h_dim), W1, b1.reshape(1, h_dim), W2,
      b2.reshape(1, h_dim), W3, b3.reshape(1, 1), cnt8, ty)


def kernel(init_features, W_init, b_init, W_agg, W_self, b_gnn,
           W1, b1, W2, b2, W3, b3, node_type, edge_index, out_idx):
    k = out_idx.shape[0]
    nt = node_type.astype(jnp.int32)
    ei = edge_index.astype(jnp.int32)
    oi = out_idx.astype(jnp.int32)
    cnt, ty = _sc_hist_gather(ei, nt, oi)
    out2d = _tc_readout(init_features.astype(jnp.float32),
                        W_init.astype(jnp.float32),
                        b_init.astype(jnp.float32),
                        W_agg.astype(jnp.float32),
                        W_self.astype(jnp.float32),
                        b_gnn.astype(jnp.float32),
                        W1.astype(jnp.float32), b1.astype(jnp.float32),
                        W2.astype(jnp.float32), b2.astype(jnp.float32),
                        W3.astype(jnp.float32), b3.astype(jnp.float32),
                        cnt, ty)
    return out2d.reshape(k)


# speedup vs baseline: 4.6905x; 1.2007x over previous
import functools

import jax
import jax.numpy as jnp
from jax import lax
from jax.experimental import pallas as pl
from jax.experimental.pallas import tpu as pltpu
from jax.experimental.pallas import tpu_sc as plsc

_NC = 2
_NS = 16
_L = 16


def _sc_hist_gather(edge_index, node_type, out_idx):
    n = node_type.shape[0]
    e = edge_index.shape[1]
    k = out_idx.shape[0]
    nw = _NC * _NS
    assert e % 128 == 0
    rows = e // 128
    base_rows = rows // nw
    extra = rows - base_rows * nw
    main_e = base_rows * 128
    chunks = base_rows + (1 if extra else 0)
    epad = chunks * 128
    assert k % _NS == 0
    kp = k // _NS
    assert kp % _L == 0
    cnt_sz = 3 * n
    cntp = -(-cnt_sz // (_NS * _L)) * (_NS * _L)
    zslice = cntp // _NS
    dump = cnt_sz

    mesh = plsc.VectorSubcoreMesh(core_axis_name="c", subcore_axis_name="s")

    @functools.partial(
        pl.kernel,
        out_type=[
            jax.ShapeDtypeStruct((_NC, 8, k), jnp.float32),
            jax.ShapeDtypeStruct((k,), jnp.int32),
        ],
        mesh=mesh,
        compiler_params=pltpu.CompilerParams(needs_layout_passes=False),
        scratch_types=[
            pltpu.VMEM((2, epad), jnp.int32),
            pltpu.VMEM((n,), jnp.int32),
            pltpu.VMEM((kp,), jnp.int32),
            pltpu.VMEM((chunks, 128), jnp.int32),
            pltpu.VMEM((128,), jnp.float32),
            pltpu.VMEM((8, 128), jnp.int32),
            pltpu.VMEM((8, 128), jnp.float32),
            pltpu.VMEM((kp,), jnp.int32),
            pltpu.VMEM((zslice,), jnp.float32),
            pltpu.VMEM_SHARED((cntp,), jnp.float32),
            pltpu.SemaphoreType.DMA,
            pltpu.SemaphoreType.DMA,
            pltpu.SemaphoreType.DMA,
        ],
    )
    def hist(edge_hbm, nt_hbm, oi_hbm, cnt_out, ty_out,
             e2_v, nt_v, oi_v, idx_e, ones_v, idxg, stg, stgt, zb,
             counts_sh, sem_in, sem_sc, sem_g):
        cid = lax.axis_index("c")
        sid = lax.axis_index("s")
        wid = cid * _NS + sid

        zeros16f = jnp.zeros((_L,), jnp.float32)
        ones16f = jnp.ones((_L,), jnp.float32)
        zeros16i = jnp.zeros((_L,), jnp.int32)
        dump16 = jnp.full((_L,), dump, jnp.int32)
        iota = lax.iota(jnp.int32, _L)

        col0 = pl.multiple_of(wid * main_e, 128)
        cp_edge = pltpu.async_copy(edge_hbm.at[:, pl.ds(col0, main_e)],
                                   e2_v.at[:, pl.ds(0, main_e)], sem_in)
        cp_nt = pltpu.async_copy(nt_hbm, nt_v, sem_in)
        cp_oi = pltpu.async_copy(oi_hbm.at[pl.ds(sid * kp, kp)], oi_v, sem_in)

        def zb_body(i, _):
            zb[pl.ds(i * _L, _L)] = zeros16f
            return 0
        lax.fori_loop(0, zslice // _L, zb_body, 0)
        pltpu.sync_copy(zb, counts_sh.at[pl.ds(sid * zslice, zslice)])

        for u in range(128 // _L):
            ones_v[pl.ds(u * _L, _L)] = ones16f

        if extra:
            for r in range(2):
                for u in range(128 // _L):
                    e2_v[r, pl.ds(main_e + u * _L, _L)] = zeros16i

            @pl.when(wid < extra)
            def _():
                tcol = pl.multiple_of((nw * base_rows + wid) * 128, 128)
                pltpu.sync_copy(edge_hbm.at[:, pl.ds(tcol, 128)],
                                e2_v.at[:, pl.ds(main_e, 128)])

        cp_edge.wait()
        cp_nt.wait()
        cp_oi.wait()

        plsc.subcore_barrier()

        DEPTH = 8

        def fire(c):
            pltpu.async_copy(ones_v, counts_sh.at[idx_e.at[c]], sem_sc,
                             add=True)

        def drain(c):
            pltpu.make_async_copy(ones_v, counts_sh.at[idx_e.at[c]],
                                  sem_sc).wait()

        def chunk_body(c, _):
            for u in range(128 // _L):
                base = c * 128 + u * _L
                s = e2_v[0, pl.ds(base, _L)]
                d = e2_v[1, pl.ds(base, _L)]
                t = plsc.load_gather(nt_v, [s])
                idx_e[c, pl.ds(u * _L, _L)] = d * 3 + t
            fire(c)

            @pl.when(c >= DEPTH)
            def _():
                drain(c - DEPTH)
            return 0
        lax.fori_loop(0, base_rows, chunk_body, 0)
        if extra:
            c = base_rows
            m = (wid < extra).astype(jnp.int32)
            im = 1 - m
            for u in range(128 // _L):
                base = c * 128 + u * _L
                s = e2_v[0, pl.ds(base, _L)]
                d = e2_v[1, pl.ds(base, _L)]
                t = plsc.load_gather(nt_v, [s])
                idx_e[c, pl.ds(u * _L, _L)] = (d * 3 + t) * m + dump16 * im
            fire(c)

        def drain_body(c, _):
            drain(c)
            return 0
        lax.fori_loop(max(0, base_rows - DEPTH), chunks, drain_body, 0)

        plsc.subcore_barrier()

        for v in range(kp // _L):
            o = plsc.load_gather(oi_v, [iota + v * _L])
            t = plsc.load_gather(nt_v, [o])
            stgt[pl.ds(v * _L, _L)] = t
            o3 = o * 3
            for j in range(8):
                if j < 3:
                    idxg[j, pl.ds(v * _L, _L)] = o3 + j
                else:
                    idxg[j, pl.ds(v * _L, _L)] = dump16
        for j in range(8):
            pltpu.async_copy(counts_sh.at[idxg.at[j]], stg.at[j], sem_g)
        for j in range(8):
            pltpu.make_async_copy(counts_sh.at[idxg.at[j]], stg.at[j],
                                  sem_g).wait()
        pltpu.sync_copy(stg, cnt_out.at[cid, :, pl.ds(sid * kp, kp)])

        @pl.when(cid == 0)
        def _():
            pltpu.sync_copy(stgt, ty_out.at[pl.ds(sid * kp, kp)])

    return hist(edge_index, node_type, out_idx)


def _tc_readout(init_features, W_init, b_init, W_agg, W_self, b_gnn,
                W1, b1, W2, b2, W3, b3, cnt8, ty):
    k = ty.shape[0]
    h_dim = W_agg.shape[0]

    def body(if_ref, wi_ref, bi_ref, wa_ref, ws_ref, bg_ref,
             w1_ref, b1_ref, w2_ref, b2_ref, w3_ref, b3_ref,
             cnt_ref, ty_ref, out_ref):
        ce_rows = [if_ref[t:t + 1, :] @ wi_ref[t] + bi_ref[t:t + 1, :]
                   for t in range(3)]
        ce8 = jnp.concatenate(ce_rows + [jnp.zeros((5, h_dim), jnp.float32)],
                              axis=0)
        m8 = ce8 @ wa_ref[...]
        s8 = ce8 @ ws_ref[...] + bg_ref[...]
        cnt = cnt_ref[0] + cnt_ref[1]
        oh = (lax.broadcasted_iota(jnp.int32, (8, k), 0)
              == ty_ref[...][None, :]).astype(jnp.float32)
        x = jnp.concatenate([cnt, oh], axis=0)
        w0 = jnp.concatenate([m8, s8], axis=0)
        h = lax.dot_general(x, w0, (((0,), (0,)), ((), ())),
                            precision=lax.Precision.HIGHEST,
                            preferred_element_type=jnp.float32)
        h = jnp.maximum(h, 0.0)
        h = jnp.maximum(h @ w1_ref[...] + b1_ref[...], 0.0)
        h = jnp.maximum(h @ w2_ref[...] + b2_ref[...], 0.0)
        z = lax.dot_general(w3_ref[...], h, (((0,), (1,)), ((), ())),
                            preferred_element_type=jnp.float32)
        out_ref[...] = jax.nn.sigmoid(z + b3_ref[...])

    return pl.pallas_call(
        body,
        out_shape=jax.ShapeDtypeStruct((1, k), jnp.float32),
    )(init_features, W_init, b_init, W_agg, W_self,
      b_gnn.reshape(1, h_dim), W1, b1.reshape(1, h_dim), W2,
      b2.reshape(1, h_dim), W3, b3.reshape(1, 1), cnt8, ty)


def kernel(init_features, W_init, b_init, W_agg, W_self, b_gnn,
           W1, b1, W2, b2, W3, b3, node_type, edge_index, out_idx):
    k = out_idx.shape[0]
    nt = node_type.astype(jnp.int32)
    ei = edge_index.astype(jnp.int32)
    oi = out_idx.astype(jnp.int32)
    cnt, ty = _sc_hist_gather(ei, nt, oi)
    out2d = _tc_readout(init_features.astype(jnp.float32),
                        W_init.astype(jnp.float32),
                        b_init.astype(jnp.float32),
                        W_agg.astype(jnp.float32),
                        W_self.astype(jnp.float32),
                        b_gnn.astype(jnp.float32),
                        W1.astype(jnp.float32), b1.astype(jnp.float32),
                        W2.astype(jnp.float32), b2.astype(jnp.float32),
                        W3.astype(jnp.float32), b3.astype(jnp.float32),
                        cnt, ty)
    return out2d.reshape(k)
